# Initial kernel scaffold; baseline (speedup 1.0000x reference)
import jax
import jax.numpy as jnp
from jax.experimental import pallas as pl


def kernel(x, pos_embedding):
    raise NotImplementedError("write your pallas kernel here")


# speedup vs baseline: 2.1418x; 2.1418x over previous
import jax
import jax.numpy as jnp
from jax.experimental import pallas as pl


def _add_body(x_ref, p_ref, o_ref):
    o_ref[...] = x_ref[...] + p_ref[...]


def kernel(x, pos_embedding):
    B, S, D = x.shape
    BS = 256
    pos = pos_embedding[:S][None]
    return pl.pallas_call(
        _add_body,
        grid=(S // BS,),
        in_specs=[
            pl.BlockSpec((B, BS, D), lambda s: (0, s, 0)),
            pl.BlockSpec((1, BS, D), lambda s: (0, s, 0)),
        ],
        out_specs=pl.BlockSpec((B, BS, D), lambda s: (0, s, 0)),
        out_shape=jax.ShapeDtypeStruct((B, S, D), x.dtype),
    )(x, pos)
